# baseline (speedup 1.0000x reference)
import functools

import jax
import jax.numpy as jnp
from jax import lax
from jax.experimental import pallas as pl
from jax.experimental.pallas import tpu as pltpu
from jax.experimental.pallas import tpu_sc as plsc

EMBED_DIM = 64
SEQ = 200
NUM_CORES = 2
NUM_SUBCORES = 16
NUM_WORKERS = NUM_CORES * NUM_SUBCORES
BPW = 128
PANELS = BPW // 16


def _pe_table(length, d):
    dim_idx = jnp.arange(d, dtype=jnp.float32)
    pos = jnp.arange(length, dtype=jnp.float32)[:, None]
    angle = pos / (10000.0 ** (2.0 * dim_idx / d))
    odd = (jnp.ones((d,), jnp.float32) - jnp.power(-1.0, dim_idx)) / 2.0
    even = jnp.ones((d,), jnp.float32) - odd
    return jnp.sin(angle) * even + jnp.cos(angle) * odd


def _rsqrt16(v):
    half = v * 0.5
    i = lax.bitcast_convert_type(v, jnp.int32)
    i = jnp.int32(0x5F3759DF) - lax.shift_right_logical(i, 1)
    y = lax.bitcast_convert_type(i, jnp.float32)
    for _ in range(2):
        y = y * (1.5 - half * y * y)
    return y


def _sc_body(table2_hbm, xt_hbm, pe_hbm, out_hbm,
             idx0, idx1, idx2a, idx2b, buf0, buf1, obuf, hbuf, pe_v,
             sem0, sem1):
    c = lax.axis_index("c")
    s = lax.axis_index("s")
    wid = s * NUM_CORES + c
    b0 = wid * BPW

    pltpu.sync_copy(pe_hbm, pe_v)
    lanes = lax.iota(jnp.int32, 16)

    def fire(l, idx_v, idx2_v, rows_v, sem):
        pltpu.sync_copy(xt_hbm.at[l, pl.ds(b0, BPW)], idx_v)
        for i in range(BPW // 16):
            idx2_v[pl.ds(16 * i, 16)] = lax.shift_right_logical(
                idx_v[pl.ds(16 * i, 16)], 1)
        pltpu.async_copy(table2_hbm.at[idx2_v], rows_v, sem)

    def drain(idx2_v, rows_v, sem):
        pltpu.make_async_copy(table2_hbm.at[idx2_v], rows_v, sem).wait()

    def compute_and_store(l, idx_v, rows_v):
        lbase = jnp.full((16,), l * EMBED_DIM, jnp.int32)

        def panel_body(p, carry2):
            row_idx = p * 16 + lanes
            off = lax.shift_left(idx_v[pl.ds(p * 16, 16)] & 1, 6)
            zero = jnp.full((16,), 0.0, jnp.float32)

            @plsc.parallel_loop(0, EMBED_DIM, 1, unroll=8,
                                carry=(zero, zero))
            def loop1(j, acc):
                sum_v, sq_v = acc
                e = plsc.load_gather(rows_v, [row_idx, off + j])
                pj = plsc.load_gather(pe_v, [lbase + j])
                h = e + pj
                hbuf[lax.shift_right_logical(j, 3),
                     pl.ds((j & 7) * 16, 16)] = h
                return (sum_v + h, sq_v + h * h)

            sum_v, sq_v = loop1
            mean_v = sum_v * (1.0 / EMBED_DIM)
            var_v = sq_v * (1.0 / EMBED_DIM) - mean_v * mean_v
            rstd_v = _rsqrt16(var_v + 1e-5)

            @plsc.parallel_loop(0, EMBED_DIM, 1, unroll=8)
            def loop2(j):
                h = hbuf[lax.shift_right_logical(j, 3),
                         pl.ds((j & 7) * 16, 16)]
                o = (h - mean_v) * rstd_v
                obuf[lax.shift_right_logical(j, 3), j & 7,
                     pl.ds(p * 16, 16)] = o
            return carry2

        lax.fori_loop(0, PANELS, panel_body, 0, unroll=False)
        pltpu.sync_copy(obuf, out_hbm.at[l, :, wid])

    fire(0, idx0, idx2a, buf0, sem0)

    def pair_body(t, carry):
        l0 = t * 2
        fire(l0 + 1, idx1, idx2b, buf1, sem1)
        drain(idx2a, buf0, sem0)
        compute_and_store(l0, idx0, buf0)

        @pl.when(l0 + 2 < SEQ)
        def _():
            fire(l0 + 2, idx0, idx2a, buf0, sem0)
        drain(idx2b, buf1, sem1)
        compute_and_store(l0 + 1, idx1, buf1)
        return carry

    lax.fori_loop(0, SEQ // 2, pair_body, 0, unroll=False)


def kernel(x, table, gamma, beta):
    batch, seq = x.shape
    table2 = table.reshape(table.shape[0] // 2, 2 * EMBED_DIM)
    xt = x.T.astype(jnp.int32)
    pe = jnp.asarray(_pe_table(seq, EMBED_DIM), jnp.float32).reshape(-1)
    del gamma, beta

    mesh = plsc.VectorSubcoreMesh(core_axis_name="c", subcore_axis_name="s")
    run = functools.partial(
        pl.kernel,
        out_type=jax.ShapeDtypeStruct((SEQ, 8, NUM_WORKERS, 8, 128),
                                      jnp.float32),
        mesh=mesh,
        compiler_params=pltpu.CompilerParams(needs_layout_passes=False),
        scratch_types=[
            pltpu.VMEM((BPW,), jnp.int32),
            pltpu.VMEM((BPW,), jnp.int32),
            pltpu.VMEM((BPW,), jnp.int32),
            pltpu.VMEM((BPW,), jnp.int32),
            pltpu.VMEM((BPW, 2 * EMBED_DIM), jnp.float32),
            pltpu.VMEM((BPW, 2 * EMBED_DIM), jnp.float32),
            pltpu.VMEM((8, 8, BPW), jnp.float32),
            pltpu.VMEM((8, BPW), jnp.float32),
            pltpu.VMEM((SEQ * EMBED_DIM,), jnp.float32),
            pltpu.SemaphoreType.DMA,
            pltpu.SemaphoreType.DMA,
        ],
    )(_sc_body)
    out = run(table2, xt, pe)
    out = out.transpose(2, 4, 0, 1, 3).reshape(batch, seq, EMBED_DIM)
    return out


# speedup vs baseline: 1.0537x; 1.0537x over previous
import functools

import jax
import jax.numpy as jnp
from jax import lax
from jax.experimental import pallas as pl
from jax.experimental.pallas import tpu as pltpu
from jax.experimental.pallas import tpu_sc as plsc

EMBED_DIM = 64
SEQ = 200
NUM_CORES = 2
NUM_SUBCORES = 16
NUM_WORKERS = NUM_CORES * NUM_SUBCORES
BPW = 128
PANELS = BPW // 16
GROUP = 8
NGROUPS = SEQ // GROUP


def _pe_table(length, d):
    dim_idx = jnp.arange(d, dtype=jnp.float32)
    pos = jnp.arange(length, dtype=jnp.float32)[:, None]
    angle = pos / (10000.0 ** (2.0 * dim_idx / d))
    odd = (jnp.ones((d,), jnp.float32) - jnp.power(-1.0, dim_idx)) / 2.0
    even = jnp.ones((d,), jnp.float32) - odd
    return jnp.sin(angle) * even + jnp.cos(angle) * odd


def _rsqrt16(v):
    half = v * 0.5
    i = lax.bitcast_convert_type(v, jnp.int32)
    i = jnp.int32(0x5F3759DF) - lax.shift_right_logical(i, 1)
    y = lax.bitcast_convert_type(i, jnp.float32)
    for _ in range(2):
        y = y * (1.5 - half * y * y)
    return y


def _sc_body(table2_hbm, xt_hbm, pe_hbm, out_hbm,
             idxbig, idx2big, r0, r1, r2, r3, o0, o1, hbuf, pe_v,
             g0, g1, g2, g3, os0, os1):
    c = lax.axis_index("c")
    s = lax.axis_index("s")
    wid = s * NUM_CORES + c
    b0 = wid * BPW

    pltpu.sync_copy(pe_hbm, pe_v)
    lanes = lax.iota(jnp.int32, 16)
    rbufs = [r0, r1, r2, r3]
    gsems = [g0, g1, g2, g3]
    obufs = [o0, o1]
    osems = [os0, os1]

    def load_group(gbase):
        pltpu.sync_copy(xt_hbm.at[pl.ds(gbase, GROUP), pl.ds(b0, BPW)],
                        idxbig)
        for r in range(GROUP):
            for i in range(BPW // 16):
                idx2big[r, pl.ds(16 * i, 16)] = lax.shift_right_logical(
                    idxbig[r, pl.ds(16 * i, 16)], 1)

    def fire(l, row, bufk):
        pltpu.async_copy(table2_hbm.at[idx2big.at[row]], rbufs[bufk],
                         gsems[bufk])

    def drain(row, bufk):
        pltpu.make_async_copy(table2_hbm.at[idx2big.at[row]], rbufs[bufk],
                              gsems[bufk]).wait()

    def out_descr(l, ob, osem):
        return pltpu.make_async_copy(ob, out_hbm.at[l, :, wid], osem)

    def compute(l, row, rows_v, ob):
        lbase = jnp.full((16,), l * EMBED_DIM, jnp.int32)

        def panel_body(p, carry2):
            row_idx = p * 16 + lanes
            off = lax.shift_left(idxbig[row, pl.ds(p * 16, 16)] & 1, 6)
            zero = jnp.full((16,), 0.0, jnp.float32)

            @plsc.parallel_loop(0, EMBED_DIM, 1, unroll=8,
                                carry=(zero, zero))
            def loop1(j, acc):
                sum_v, sq_v = acc
                e = plsc.load_gather(rows_v, [row_idx, off + j])
                pj = plsc.load_gather(pe_v, [lbase + j])
                h = e + pj
                hbuf[lax.shift_right_logical(j, 3),
                     pl.ds((j & 7) * 16, 16)] = h
                return (sum_v + h, sq_v + h * h)

            sum_v, sq_v = loop1
            mean_v = sum_v * (1.0 / EMBED_DIM)
            var_v = sq_v * (1.0 / EMBED_DIM) - mean_v * mean_v
            rstd_v = _rsqrt16(var_v + 1e-5)

            @plsc.parallel_loop(0, EMBED_DIM, 1, unroll=8)
            def loop2(j):
                h = hbuf[lax.shift_right_logical(j, 3),
                         pl.ds((j & 7) * 16, 16)]
                o = (h - mean_v) * rstd_v
                ob[lax.shift_right_logical(j, 3), j & 7,
                   pl.ds(p * 16, 16)] = o
            return carry2

        lax.fori_loop(0, PANELS, panel_body, 0, unroll=False)

    load_group(0)
    fire(0, 0, 0)
    fire(1, 1, 1)
    fire(2, 2, 2)

    def group_body(g, carry):
        gbase = g * GROUP

        @pl.when(g > 0)
        def _():
            load_group(gbase)
            fire(gbase, 0, 0)
            fire(gbase + 1, 1, 1)
            fire(gbase + 2, 2, 2)

        def chunk_body(u, carry2):
            for k in range(4):
                row = u * 4 + k
                l = gbase + row
                drain(row, k)

                @pl.when(l >= 2)
                def _():
                    out_descr(l, obufs[k % 2], osems[k % 2]).wait()
                compute(l, row, rbufs[k], obufs[k % 2])
                out_descr(l, obufs[k % 2], osems[k % 2]).start()

                @pl.when(row + 3 < GROUP)
                def _():
                    fire(l + 3, row + 3, (k + 3) % 4)
            return carry2

        lax.fori_loop(0, GROUP // 4, chunk_body, 0, unroll=False)
        return carry

    lax.fori_loop(0, NGROUPS, group_body, 0, unroll=False)
    out_descr(SEQ - 2, obufs[0], osems[0]).wait()
    out_descr(SEQ - 1, obufs[1], osems[1]).wait()


def kernel(x, table, gamma, beta):
    batch, seq = x.shape
    table2 = table.reshape(table.shape[0] // 2, 2 * EMBED_DIM)
    xt = x.T.astype(jnp.int32)
    pe = jnp.asarray(_pe_table(seq, EMBED_DIM), jnp.float32).reshape(-1)
    del gamma, beta

    mesh = plsc.VectorSubcoreMesh(core_axis_name="c", subcore_axis_name="s")
    run = functools.partial(
        pl.kernel,
        out_type=jax.ShapeDtypeStruct((SEQ, 8, NUM_WORKERS, 8, 128),
                                      jnp.float32),
        mesh=mesh,
        compiler_params=pltpu.CompilerParams(needs_layout_passes=False),
        scratch_types=[
            pltpu.VMEM((GROUP, BPW), jnp.int32),
            pltpu.VMEM((GROUP, BPW), jnp.int32),
            pltpu.VMEM((BPW, 2 * EMBED_DIM), jnp.float32),
            pltpu.VMEM((BPW, 2 * EMBED_DIM), jnp.float32),
            pltpu.VMEM((BPW, 2 * EMBED_DIM), jnp.float32),
            pltpu.VMEM((BPW, 2 * EMBED_DIM), jnp.float32),
            pltpu.VMEM((8, 8, BPW), jnp.float32),
            pltpu.VMEM((8, 8, BPW), jnp.float32),
            pltpu.VMEM((8, BPW), jnp.float32),
            pltpu.VMEM((SEQ * EMBED_DIM,), jnp.float32),
            pltpu.SemaphoreType.DMA,
            pltpu.SemaphoreType.DMA,
            pltpu.SemaphoreType.DMA,
            pltpu.SemaphoreType.DMA,
            pltpu.SemaphoreType.DMA,
            pltpu.SemaphoreType.DMA,
        ],
    )(_sc_body)
    out = run(table2, xt, pe)
    out = out.transpose(2, 4, 0, 1, 3).reshape(batch, seq, EMBED_DIM)
    return out
